# baseline (speedup 1.0000x reference)
import functools

import jax
import jax.numpy as jnp
from jax import lax
from jax.experimental import pallas as pl
from jax.experimental.pallas import tpu as pltpu
from jax.experimental.pallas import tpu_sc as plsc

N = 10000
E = 320000
F_IN = 128
H = 256
B = 64
D = 16
EPS = 1e-5

BLK = 1000
NB = N // BLK
NC = 2
NS = 16
CHUNK = 128
EP = 327680
PE = 5120
ZROWS = 624
ZTAIL = N - NS * ZROWS


def _sc_scatter_body(split_features, table, srcx, dst3, zeros, out, *refs):
    src_all, dst_all, rows0, rows1, acc, gs0, gs1, ss0, ss1 = refs
    rows = (rows0, rows1)
    gsem = (gs0, gs1)
    ssem = (ss0, ss1)
    c = lax.axis_index("c")
    s = lax.axis_index("s")
    r0 = s * ZROWS
    pltpu.sync_copy(zeros.at[pl.ds(0, ZROWS)], acc.at[pl.ds(r0, ZROWS)])

    @pl.when(s == 0)
    def _():
        pltpu.sync_copy(zeros.at[pl.ds(0, ZTAIL)],
                        acc.at[pl.ds(NS * ZROWS, ZTAIL)])

    plsc.subcore_barrier()

    if split_features:
        ne = EP // NS
        src_base = c * EP + s * ne
        dst_base = s * ne
    else:
        ne = EP // (NC * NS)
        w = s * NC + c
        src_base = w * ne
        dst_base = w * ne
    nphase = ne // PE
    nchp = PE // CHUNK

    def gfire(b, j):
        pltpu.async_copy(
            table.at[src_all.at[pl.ds(j * CHUNK, CHUNK)]], rows[b], gsem[b])

    def gwait(b):
        pltpu.make_async_copy(
            table.at[src_all.at[pl.ds(0, CHUNK)]], rows[b], gsem[b]).wait()

    def sfire(b, j):
        pltpu.async_copy(rows[b], acc.at[dst_all.at[j, 0]], ssem[b], add=True)

    def swait(b):
        pltpu.make_async_copy(rows[b], acc.at[dst_all.at[0, 0]], ssem[b]).wait()

    for p in range(nphase):
        pltpu.sync_copy(srcx.at[pl.ds(src_base + p * PE, PE)], src_all)
        pltpu.sync_copy(
            dst3.at[pl.ds((dst_base + p * PE) // CHUNK, nchp)], dst_all)
        gfire(0, 0)
        gfire(1, 1)

        def body(t, carry):
            j0 = 2 * t
            gwait(0)
            sfire(0, j0)
            gwait(1)
            sfire(1, j0 + 1)
            swait(0)
            gfire(0, j0 + 2)
            swait(1)
            gfire(1, j0 + 3)
            return carry

        lax.fori_loop(0, nchp // 2 - 1, body, 0)
        gwait(0)
        sfire(0, nchp - 2)
        gwait(1)
        sfire(1, nchp - 1)
        swait(0)
        swait(1)
    plsc.subcore_barrier()
    pltpu.sync_copy(acc.at[pl.ds(r0, ZROWS)],
                    out.at[pl.ds(c * N + r0, ZROWS)])

    @pl.when(s == 0)
    def _():
        pltpu.sync_copy(acc.at[pl.ds(NS * ZROWS, ZTAIL)],
                        out.at[pl.ds(c * N + NS * ZROWS, ZTAIL)])


def _make_sc_scatter(split_features):
    mesh = plsc.VectorSubcoreMesh(core_axis_name="c", subcore_axis_name="s",
                                  num_cores=NC, num_subcores=NS)
    return pl.kernel(
        functools.partial(_sc_scatter_body, split_features),
        out_type=jax.ShapeDtypeStruct((NC * N, 128), jnp.float32),
        mesh=mesh,
        scratch_types=(
            [pltpu.VMEM((PE,), jnp.int32),
             pltpu.VMEM((PE // CHUNK, 1, CHUNK), jnp.int32),
             pltpu.VMEM((CHUNK, 128), jnp.float32),
             pltpu.VMEM((CHUNK, 128), jnp.float32),
             pltpu.VMEM_SHARED((N + 8, 128), jnp.float32)]
            + [pltpu.SemaphoreType.DMA for _ in range(4)]
        ),
    )


def _seg_stats(i, u, b_vec, st_ref):
    oh = (b_vec[None, :] == lax.broadcasted_iota(jnp.int32, (B, BLK), 0)
          ).astype(jnp.float32)
    r1 = jnp.sum(u, axis=1)
    r2 = jnp.sum(u * u, axis=1)

    @pl.when(i == 0)
    def _():
        st_ref[...] = jnp.zeros_like(st_ref)

    st_ref[0, :] += jnp.sum(oh * r1[None, :], axis=1)
    st_ref[1, :] += jnp.sum(oh * r2[None, :], axis=1)
    st_ref[2, :] += jnp.sum(oh, axis=1)


def _t1a_body(x_ref, p0_ref, p1_ref, b_ref, Wa_ref, ba_ref, Wb_ref, bb_ref,
              u_ref, st_ref):
    i = pl.program_id(0)
    y = x_ref[...] + p0_ref[...] + p1_ref[...]
    t = jnp.dot(y, Wa_ref[...], preferred_element_type=jnp.float32)
    t = jnp.maximum(t + ba_ref[...][None, :], 0.0)
    u = jnp.dot(t, Wb_ref[...], preferred_element_type=jnp.float32)
    u = u + bb_ref[...][None, :]
    u_ref[...] = u
    _seg_stats(i, u, b_ref[0, 0], st_ref)


def _t2a_body(h0_ref, h1_ref, a0_ref, a1_ref, b_ref, Wat_ref, Wab_ref,
              ba_ref, Wb_ref, bb_ref, u_ref, st_ref):
    i = pl.program_id(0)
    y0 = h0_ref[...] + a0_ref[...]
    y1 = h1_ref[...] + a1_ref[...]
    t = (jnp.dot(y0, Wat_ref[...], preferred_element_type=jnp.float32)
         + jnp.dot(y1, Wab_ref[...], preferred_element_type=jnp.float32))
    t = jnp.maximum(t + ba_ref[...][None, :], 0.0)
    u = jnp.dot(t, Wb_ref[...], preferred_element_type=jnp.float32)
    u = u + bb_ref[...][None, :]
    u_ref[...] = u
    _seg_stats(i, u, b_ref[0, 0], st_ref)


def _graph_ln(u, b_vec, st_ref, w_ref, bias_ref):
    deg = st_ref[2, :]
    norm = jnp.maximum(deg, 1.0) * H
    mean_g = st_ref[0, :] / norm
    var_g = st_ref[1, :] / norm - mean_g * mean_g
    inv_g = 1.0 / jnp.sqrt(var_g + EPS)
    ohT = (b_vec[:, None] == lax.broadcasted_iota(jnp.int32, (BLK, B), 1)
           ).astype(jnp.float32)
    mean_n = jnp.dot(ohT, mean_g[:, None], preferred_element_type=jnp.float32,
                 precision=lax.Precision.HIGHEST)
    inv_n = jnp.dot(ohT, inv_g[:, None], preferred_element_type=jnp.float32,
                 precision=lax.Precision.HIGHEST)
    out = (u - mean_n) * inv_n * w_ref[...][None, :] + bias_ref[...][None, :]
    return jnp.maximum(out, 0.0)


def _t1b_body(u_ref, st_ref, b_ref, w_ref, bias_ref, h_ref):
    out = _graph_ln(u_ref[...], b_ref[0, 0], st_ref, w_ref, bias_ref)
    h_ref[0] = out[:, :128]
    h_ref[1] = out[:, 128:]


def _t2b_body(u_ref, st_ref, b_ref, w_ref, bias_ref, add_ref):
    i = pl.program_id(0)
    out = _graph_ln(u_ref[...], b_ref[0, 0], st_ref, w_ref, bias_ref)
    b_vec = b_ref[0, 0]
    oh = (b_vec[None, :] == lax.broadcasted_iota(jnp.int32, (B, BLK), 0)
          ).astype(jnp.float32)

    @pl.when(i == 0)
    def _():
        add_ref[...] = jnp.zeros_like(add_ref)

    add_ref[...] += jnp.dot(oh, out, preferred_element_type=jnp.float32,
                 precision=lax.Precision.HIGHEST)


def _t3_body(a1_ref, st1_ref, a2_ref, st2_ref, d1_ref, d2_ref,
             nw1, nw2, nw3, nw4, nb1, nb2, nb3, nb4,
             f1A, f1B, f1C, f1D, f1b_ref, f2W_ref, f2b_ref,
             oW_ref, ob_ref, out_ref):
    IN = 2 * H + 2 * D
    deg1 = jnp.maximum(st1_ref[2, :], 1.0)[:, None]
    emb1 = a1_ref[...] * (1.0 + 1.0 / deg1)
    deg2 = jnp.maximum(st2_ref[2, :], 1.0)[:, None]
    emb2 = a2_ref[...] * (1.0 + 1.0 / deg2)
    p1 = emb1 + emb2
    p2 = jnp.abs(emb1 - emb2)
    p3 = d1_ref[...] + d2_ref[...]
    p4 = jnp.abs(d1_ref[...] - d2_ref[...])
    s = (jnp.sum(p1, axis=1) + jnp.sum(p2, axis=1)
         + jnp.sum(p3, axis=1) + jnp.sum(p4, axis=1))
    mu = s / IN
    q = (jnp.sum(p1 * p1, axis=1) + jnp.sum(p2 * p2, axis=1)
         + jnp.sum(p3 * p3, axis=1) + jnp.sum(p4 * p4, axis=1))
    var = q / IN - mu * mu
    inv = (1.0 / jnp.sqrt(var + EPS))[:, None]
    mu = mu[:, None]
    c1 = (p1 - mu) * inv * nw1[...][None, :] + nb1[...][None, :]
    c2 = (p2 - mu) * inv * nw2[...][None, :] + nb2[...][None, :]
    c3 = (p3 - mu) * inv * nw3[...][None, :] + nb3[...][None, :]
    c4 = (p4 - mu) * inv * nw4[...][None, :] + nb4[...][None, :]
    z = (jnp.dot(c1, f1A[...], preferred_element_type=jnp.float32)
         + jnp.dot(c2, f1B[...], preferred_element_type=jnp.float32)
         + jnp.dot(c3, f1C[...], preferred_element_type=jnp.float32)
         + jnp.dot(c4, f1D[...], preferred_element_type=jnp.float32))
    z = jnp.maximum(z + f1b_ref[...][None, :], 0.0)
    z = jnp.dot(z, f2W_ref[...], preferred_element_type=jnp.float32)
    z = jnp.maximum(z + f2b_ref[...][None, :], 0.0)
    z = jnp.dot(z, oW_ref[...], preferred_element_type=jnp.float32)
    out_ref[...] = z + ob_ref[...][None, :]


def _build():
    fns = {}

    def row_blk(off):
        return pl.BlockSpec((BLK, 128), lambda i, off=off: (off + i, 0))

    batch_spec = pl.BlockSpec((1, 1, BLK), lambda i: (i, 0, 0))

    def full(shape):
        return pl.BlockSpec(shape, lambda i, n=len(shape): (0,) * n)

    st_spec = pl.BlockSpec((8, B), lambda i: (0, 0))

    fns["t1a"] = pl.pallas_call(
        _t1a_body,
        grid=(NB,),
        in_specs=[
            pl.BlockSpec((BLK, F_IN), lambda i: (i, 0)),
            row_blk(0), row_blk(NB),
            batch_spec,
            full((F_IN, H)), full((H,)), full((H, H)), full((H,)),
        ],
        out_specs=(pl.BlockSpec((BLK, H), lambda i: (i, 0)), st_spec),
        out_shape=(jax.ShapeDtypeStruct((N, H), jnp.float32),
                   jax.ShapeDtypeStruct((8, B), jnp.float32)),
    )
    fns["t2a"] = pl.pallas_call(
        _t2a_body,
        grid=(NB,),
        in_specs=[
            row_blk(0), row_blk(NB), row_blk(0), row_blk(NB),
            batch_spec,
            pl.BlockSpec((128, H), lambda i: (0, 0)),
            pl.BlockSpec((128, H), lambda i: (1, 0)),
            full((H,)), full((H, H)), full((H,)),
        ],
        out_specs=(pl.BlockSpec((BLK, H), lambda i: (i, 0)), st_spec),
        out_shape=(jax.ShapeDtypeStruct((N, H), jnp.float32),
                   jax.ShapeDtypeStruct((8, B), jnp.float32)),
    )
    fns["t1b"] = pl.pallas_call(
        _t1b_body,
        grid=(NB,),
        in_specs=[
            pl.BlockSpec((BLK, H), lambda i: (i, 0)),
            st_spec, batch_spec, full((H,)), full((H,)),
        ],
        out_specs=pl.BlockSpec((2, BLK, 128), lambda i: (0, i, 0)),
        out_shape=jax.ShapeDtypeStruct((2, N, 128), jnp.float32),
    )
    fns["t2b"] = pl.pallas_call(
        _t2b_body,
        grid=(NB,),
        in_specs=[
            pl.BlockSpec((BLK, H), lambda i: (i, 0)),
            st_spec, batch_spec, full((H,)), full((H,)),
        ],
        out_specs=pl.BlockSpec((B, H), lambda i: (0, 0)),
        out_shape=jax.ShapeDtypeStruct((B, H), jnp.float32),
    )
    fns["t3"] = pl.pallas_call(
        _t3_body,
        out_shape=jax.ShapeDtypeStruct((B, 1), jnp.float32),
    )
    return fns


_FNS = _build()


def _get_sc(name):
    fn = _FNS.get(name)
    if fn is None:
        fn = _FNS[name] = _make_sc_scatter(name == "sc2")
    return fn


def _backbone(x, src, dst, batch3, zeros,
              W1a, b1a, W1b, b1b, ln1_w, ln1_b,
              W2a, b2a, W2b, b2b, ln2_w, ln2_b):
    pad = jnp.zeros((EP - E,), jnp.int32)
    srcp = jnp.concatenate([src, pad], axis=0)
    dst3 = jnp.concatenate([dst, pad + N], axis=0).reshape(EP // CHUNK, 1, CHUNK)
    p = _get_sc("sc1")(x, srcp, dst3, zeros)
    u1, st1 = _FNS["t1a"](x, p, p, batch3, W1a, b1a, W1b, b1b)
    h = _FNS["t1b"](u1, st1, batch3, ln1_w, ln1_b)
    h2 = h.reshape(NC * N, 128)
    src2 = jnp.concatenate([srcp, srcp + N], axis=0)
    a = _get_sc("sc2")(h2, src2, dst3, zeros)
    u2, st2 = _FNS["t2a"](h2, h2, a, a, batch3, W2a, W2a, b2a, W2b, b2b)
    add = _FNS["t2b"](u2, st2, batch3, ln2_w, ln2_b)
    return add, st2


def kernel(g1_x, g1_edge_index, g1_batch, g2_x, g2_edge_index, g2_batch,
           d1, d2, W1a, b1a, W1b, b1b, ln1_w, ln1_b, W2a, b2a, W2b, b2b,
           ln2_w, ln2_b, norm_w, norm_b, fc1_W, fc1_b, fc2_W, fc2_b,
           out_W, out_b):
    zeros = jnp.zeros((ZROWS, 128), jnp.float32)
    bb = (W1a, b1a, W1b, b1b, ln1_w, ln1_b, W2a, b2a, W2b, b2b, ln2_w, ln2_b)
    add1, st1 = _backbone(g1_x, g1_edge_index[0], g1_edge_index[1],
                          g1_batch.reshape(NB, 1, BLK), zeros, *bb)
    add2, st2 = _backbone(g2_x, g2_edge_index[0], g2_edge_index[1],
                          g2_batch.reshape(NB, 1, BLK), zeros, *bb)
    return _FNS["t3"](
        add1, st1, add2, st2, d1, d2,
        norm_w[:H], norm_w[H:2 * H], norm_w[2 * H:2 * H + D], norm_w[2 * H + D:],
        norm_b[:H], norm_b[H:2 * H], norm_b[2 * H:2 * H + D], norm_b[2 * H + D:],
        fc1_W[:H], fc1_W[H:2 * H], fc1_W[2 * H:2 * H + D], fc1_W[2 * H + D:],
        fc1_b, fc2_W, fc2_b, out_W, out_b)


# speedup vs baseline: 2.5370x; 2.5370x over previous
import functools

import jax
import jax.numpy as jnp
from jax import lax
from jax.experimental import pallas as pl
from jax.experimental.pallas import tpu as pltpu
from jax.experimental.pallas import tpu_sc as plsc

N = 10000
E = 320000
F_IN = 128
H = 256
B = 64
D = 16
EPS = 1e-5

BLK = 1000
NB = N // BLK
NC = 2
NS = 16
CHUNK = 80
NBUF = 4
ZROWS = 624
ZTAIL = N - NS * ZROWS


def _sc_scatter_body(split_features, table, srcx, dst, zeros, out, *refs):
    sv = refs[0:NBUF]
    dv = refs[NBUF:2 * NBUF]
    rows = refs[2 * NBUF:3 * NBUF]
    acc = refs[3 * NBUF]
    isem = refs[3 * NBUF + 1:3 * NBUF + 3]
    gsem = refs[3 * NBUF + 3:3 * NBUF + 5]
    ssem = refs[3 * NBUF + 5:3 * NBUF + 7]
    c = lax.axis_index("c")
    s = lax.axis_index("s")
    r0 = s * ZROWS
    pltpu.sync_copy(zeros.at[pl.ds(0, ZROWS)], acc.at[pl.ds(r0, ZROWS)])

    @pl.when(s == 0)
    def _():
        pltpu.sync_copy(zeros.at[pl.ds(0, ZTAIL)],
                        acc.at[pl.ds(NS * ZROWS, ZTAIL)])

    plsc.subcore_barrier()

    if split_features:
        ne = E // NS
        src_base = c * E + s * ne
        dst_base = s * ne
    else:
        ne = E // (NC * NS)
        w = s * NC + c
        src_base = w * ne
        dst_base = w * ne
    nch = ne // CHUNK
    ngrp = nch // NBUF
    tail = nch % NBUF
    HALF = NBUF // 2

    def half_step(h, j0, k, wait_prev):
        bs = list(range(h * HALF, h * HALF + k))
        if wait_prev:
            for b in range(h * HALF, (h + 1) * HALF):
                pltpu.make_async_copy(rows[b], acc.at[dv[b]], ssem[h]).wait()
        for i, b in enumerate(bs):
            pltpu.async_copy(
                srcx.at[pl.ds(src_base + (j0 + i) * CHUNK, CHUNK)], sv[b], isem[h])
            pltpu.async_copy(
                dst.at[pl.ds(dst_base + (j0 + i) * CHUNK, CHUNK)], dv[b], isem[h])
        for b in bs:
            pltpu.make_async_copy(srcx.at[pl.ds(src_base, CHUNK)], sv[b], isem[h]).wait()
            pltpu.make_async_copy(dst.at[pl.ds(dst_base, CHUNK)], dv[b], isem[h]).wait()
        for b in bs:
            pltpu.async_copy(table.at[sv[b]], rows[b], gsem[h])
        for b in bs:
            pltpu.make_async_copy(table.at[sv[b]], rows[b], gsem[h]).wait()
        for b in bs:
            pltpu.async_copy(rows[b], acc.at[dv[b]], ssem[h], add=True)

    half_step(0, 0, HALF, False)
    half_step(1, HALF, HALF, False)

    def body(g, carry):
        j0 = g * NBUF
        half_step(0, j0, HALF, True)
        half_step(1, j0 + HALF, HALF, True)
        return carry

    lax.fori_loop(1, ngrp, body, 0)
    if tail:
        half_step(0, ngrp * NBUF, tail, True)
    for b in range(tail if tail else HALF):
        pltpu.make_async_copy(rows[b], acc.at[dv[b]], ssem[0]).wait()
    for b in range(HALF, NBUF):
        pltpu.make_async_copy(rows[b], acc.at[dv[b]], ssem[1]).wait()
    plsc.subcore_barrier()
    pltpu.sync_copy(acc.at[pl.ds(r0, ZROWS)],
                    out.at[pl.ds(c * N + r0, ZROWS)])

    @pl.when(s == 0)
    def _():
        pltpu.sync_copy(acc.at[pl.ds(NS * ZROWS, ZTAIL)],
                        out.at[pl.ds(c * N + NS * ZROWS, ZTAIL)])


def _make_sc_scatter(split_features):
    mesh = plsc.VectorSubcoreMesh(core_axis_name="c", subcore_axis_name="s",
                                  num_cores=NC, num_subcores=NS)
    return pl.kernel(
        functools.partial(_sc_scatter_body, split_features),
        out_type=jax.ShapeDtypeStruct((NC * N, 128), jnp.float32),
        mesh=mesh,
        scratch_types=(
            [pltpu.VMEM((CHUNK,), jnp.int32) for _ in range(2 * NBUF)]
            + [pltpu.VMEM((CHUNK, 128), jnp.float32) for _ in range(NBUF)]
            + [pltpu.VMEM_SHARED((N, 128), jnp.float32)]
            + [pltpu.SemaphoreType.DMA for _ in range(6)]
        ),
    )


def _seg_stats(i, u, b_vec, st_ref):
    oh = (b_vec[None, :] == lax.broadcasted_iota(jnp.int32, (B, BLK), 0)
          ).astype(jnp.float32)
    r1 = jnp.sum(u, axis=1)
    r2 = jnp.sum(u * u, axis=1)

    @pl.when(i == 0)
    def _():
        st_ref[...] = jnp.zeros_like(st_ref)

    st_ref[0, :] += jnp.sum(oh * r1[None, :], axis=1)
    st_ref[1, :] += jnp.sum(oh * r2[None, :], axis=1)
    st_ref[2, :] += jnp.sum(oh, axis=1)


def _t1a_body(x_ref, p0_ref, p1_ref, b_ref, Wa_ref, ba_ref, Wb_ref, bb_ref,
              u_ref, st_ref):
    i = pl.program_id(0)
    y = x_ref[...] + p0_ref[...] + p1_ref[...]
    t = jnp.dot(y, Wa_ref[...], preferred_element_type=jnp.float32)
    t = jnp.maximum(t + ba_ref[...][None, :], 0.0)
    u = jnp.dot(t, Wb_ref[...], preferred_element_type=jnp.float32)
    u = u + bb_ref[...][None, :]
    u_ref[...] = u
    _seg_stats(i, u, b_ref[0, 0], st_ref)


def _t2a_body(h0_ref, h1_ref, a0_ref, a1_ref, b_ref, Wat_ref, Wab_ref,
              ba_ref, Wb_ref, bb_ref, u_ref, st_ref):
    i = pl.program_id(0)
    y0 = h0_ref[...] + a0_ref[...]
    y1 = h1_ref[...] + a1_ref[...]
    t = (jnp.dot(y0, Wat_ref[...], preferred_element_type=jnp.float32)
         + jnp.dot(y1, Wab_ref[...], preferred_element_type=jnp.float32))
    t = jnp.maximum(t + ba_ref[...][None, :], 0.0)
    u = jnp.dot(t, Wb_ref[...], preferred_element_type=jnp.float32)
    u = u + bb_ref[...][None, :]
    u_ref[...] = u
    _seg_stats(i, u, b_ref[0, 0], st_ref)


def _graph_ln(u, b_vec, st_ref, w_ref, bias_ref):
    deg = st_ref[2, :]
    norm = jnp.maximum(deg, 1.0) * H
    mean_g = st_ref[0, :] / norm
    var_g = st_ref[1, :] / norm - mean_g * mean_g
    inv_g = 1.0 / jnp.sqrt(var_g + EPS)
    ohT = (b_vec[:, None] == lax.broadcasted_iota(jnp.int32, (BLK, B), 1)
           ).astype(jnp.float32)
    mean_n = jnp.dot(ohT, mean_g[:, None], preferred_element_type=jnp.float32,
                     precision=lax.Precision.HIGHEST)
    inv_n = jnp.dot(ohT, inv_g[:, None], preferred_element_type=jnp.float32,
                    precision=lax.Precision.HIGHEST)
    out = (u - mean_n) * inv_n * w_ref[...][None, :] + bias_ref[...][None, :]
    return jnp.maximum(out, 0.0)


def _t1b_body(u_ref, st_ref, b_ref, w_ref, bias_ref, h_ref):
    out = _graph_ln(u_ref[...], b_ref[0, 0], st_ref, w_ref, bias_ref)
    h_ref[0] = out[:, :128]
    h_ref[1] = out[:, 128:]


def _t2b_body(u_ref, st_ref, b_ref, w_ref, bias_ref, add_ref):
    i = pl.program_id(0)
    out = _graph_ln(u_ref[...], b_ref[0, 0], st_ref, w_ref, bias_ref)
    b_vec = b_ref[0, 0]
    oh = (b_vec[None, :] == lax.broadcasted_iota(jnp.int32, (B, BLK), 0)
          ).astype(jnp.float32)

    @pl.when(i == 0)
    def _():
        add_ref[...] = jnp.zeros_like(add_ref)

    add_ref[...] += jnp.dot(oh, out, preferred_element_type=jnp.float32,
                            precision=lax.Precision.HIGHEST)


def _t3_body(a1_ref, st1_ref, a2_ref, st2_ref, d1_ref, d2_ref,
             nw1, nw2, nw3, nw4, nb1, nb2, nb3, nb4,
             f1A, f1B, f1C, f1D, f1b_ref, f2W_ref, f2b_ref,
             oW_ref, ob_ref, out_ref):
    IN = 2 * H + 2 * D
    deg1 = jnp.maximum(st1_ref[2, :], 1.0)[:, None]
    emb1 = a1_ref[...] * (1.0 + 1.0 / deg1)
    deg2 = jnp.maximum(st2_ref[2, :], 1.0)[:, None]
    emb2 = a2_ref[...] * (1.0 + 1.0 / deg2)
    p1 = emb1 + emb2
    p2 = jnp.abs(emb1 - emb2)
    p3 = d1_ref[...] + d2_ref[...]
    p4 = jnp.abs(d1_ref[...] - d2_ref[...])
    s = (jnp.sum(p1, axis=1) + jnp.sum(p2, axis=1)
         + jnp.sum(p3, axis=1) + jnp.sum(p4, axis=1))
    mu = s / IN
    q = (jnp.sum(p1 * p1, axis=1) + jnp.sum(p2 * p2, axis=1)
         + jnp.sum(p3 * p3, axis=1) + jnp.sum(p4 * p4, axis=1))
    var = q / IN - mu * mu
    inv = (1.0 / jnp.sqrt(var + EPS))[:, None]
    mu = mu[:, None]
    c1 = (p1 - mu) * inv * nw1[...][None, :] + nb1[...][None, :]
    c2 = (p2 - mu) * inv * nw2[...][None, :] + nb2[...][None, :]
    c3 = (p3 - mu) * inv * nw3[...][None, :] + nb3[...][None, :]
    c4 = (p4 - mu) * inv * nw4[...][None, :] + nb4[...][None, :]
    z = (jnp.dot(c1, f1A[...], preferred_element_type=jnp.float32)
         + jnp.dot(c2, f1B[...], preferred_element_type=jnp.float32)
         + jnp.dot(c3, f1C[...], preferred_element_type=jnp.float32)
         + jnp.dot(c4, f1D[...], preferred_element_type=jnp.float32))
    z = jnp.maximum(z + f1b_ref[...][None, :], 0.0)
    z = jnp.dot(z, f2W_ref[...], preferred_element_type=jnp.float32)
    z = jnp.maximum(z + f2b_ref[...][None, :], 0.0)
    z = jnp.dot(z, oW_ref[...], preferred_element_type=jnp.float32)
    out_ref[...] = z + ob_ref[...][None, :]


def _build():
    fns = {}

    def row_blk(off):
        return pl.BlockSpec((BLK, 128), lambda i, off=off: (off + i, 0))

    batch_spec = pl.BlockSpec((1, 1, BLK), lambda i: (i, 0, 0))

    def full(shape):
        return pl.BlockSpec(shape, lambda i, n=len(shape): (0,) * n)

    st_spec = pl.BlockSpec((8, B), lambda i: (0, 0))

    fns["t1a"] = pl.pallas_call(
        _t1a_body,
        grid=(NB,),
        in_specs=[
            pl.BlockSpec((BLK, F_IN), lambda i: (i, 0)),
            row_blk(0), row_blk(NB),
            batch_spec,
            full((F_IN, H)), full((H,)), full((H, H)), full((H,)),
        ],
        out_specs=(pl.BlockSpec((BLK, H), lambda i: (i, 0)), st_spec),
        out_shape=(jax.ShapeDtypeStruct((N, H), jnp.float32),
                   jax.ShapeDtypeStruct((8, B), jnp.float32)),
    )
    fns["t2a"] = pl.pallas_call(
        _t2a_body,
        grid=(NB,),
        in_specs=[
            row_blk(0), row_blk(NB), row_blk(0), row_blk(NB),
            batch_spec,
            pl.BlockSpec((128, H), lambda i: (0, 0)),
            pl.BlockSpec((128, H), lambda i: (1, 0)),
            full((H,)), full((H, H)), full((H,)),
        ],
        out_specs=(pl.BlockSpec((BLK, H), lambda i: (i, 0)), st_spec),
        out_shape=(jax.ShapeDtypeStruct((N, H), jnp.float32),
                   jax.ShapeDtypeStruct((8, B), jnp.float32)),
    )
    fns["t1b"] = pl.pallas_call(
        _t1b_body,
        grid=(NB,),
        in_specs=[
            pl.BlockSpec((BLK, H), lambda i: (i, 0)),
            st_spec, batch_spec, full((H,)), full((H,)),
        ],
        out_specs=pl.BlockSpec((2, BLK, 128), lambda i: (0, i, 0)),
        out_shape=jax.ShapeDtypeStruct((2, N, 128), jnp.float32),
    )
    fns["t2b"] = pl.pallas_call(
        _t2b_body,
        grid=(NB,),
        in_specs=[
            pl.BlockSpec((BLK, H), lambda i: (i, 0)),
            st_spec, batch_spec, full((H,)), full((H,)),
        ],
        out_specs=pl.BlockSpec((B, H), lambda i: (0, 0)),
        out_shape=jax.ShapeDtypeStruct((B, H), jnp.float32),
    )
    fns["t3"] = pl.pallas_call(
        _t3_body,
        out_shape=jax.ShapeDtypeStruct((B, 1), jnp.float32),
    )
    return fns


_FNS = _build()


def _get_sc(name):
    fn = _FNS.get(name)
    if fn is None:
        fn = _FNS[name] = _make_sc_scatter(name == "sc2")
    return fn


def _backbone(x, src, dst, batch3, zeros,
              W1a, b1a, W1b, b1b, ln1_w, ln1_b,
              W2a, b2a, W2b, b2b, ln2_w, ln2_b):
    p = _get_sc("sc1")(x, src, dst, zeros)
    u1, st1 = _FNS["t1a"](x, p, p, batch3, W1a, b1a, W1b, b1b)
    h = _FNS["t1b"](u1, st1, batch3, ln1_w, ln1_b)
    h2 = h.reshape(NC * N, 128)
    src2 = jnp.concatenate([src, src + N], axis=0)
    a = _get_sc("sc2")(h2, src2, dst, zeros)
    u2, st2 = _FNS["t2a"](h2, h2, a, a, batch3, W2a, W2a, b2a, W2b, b2b)
    add = _FNS["t2b"](u2, st2, batch3, ln2_w, ln2_b)
    return add, st2


def kernel(g1_x, g1_edge_index, g1_batch, g2_x, g2_edge_index, g2_batch,
           d1, d2, W1a, b1a, W1b, b1b, ln1_w, ln1_b, W2a, b2a, W2b, b2b,
           ln2_w, ln2_b, norm_w, norm_b, fc1_W, fc1_b, fc2_W, fc2_b,
           out_W, out_b):
    zeros = jnp.zeros((ZROWS, 128), jnp.float32)
    bb = (W1a, b1a, W1b, b1b, ln1_w, ln1_b, W2a, b2a, W2b, b2b, ln2_w, ln2_b)
    add1, st1 = _backbone(g1_x, g1_edge_index[0], g1_edge_index[1],
                          g1_batch.reshape(NB, 1, BLK), zeros, *bb)
    add2, st2 = _backbone(g2_x, g2_edge_index[0], g2_edge_index[1],
                          g2_batch.reshape(NB, 1, BLK), zeros, *bb)
    return _FNS["t3"](
        add1, st1, add2, st2, d1, d2,
        norm_w[:H], norm_w[H:2 * H], norm_w[2 * H:2 * H + D], norm_w[2 * H + D:],
        norm_b[:H], norm_b[H:2 * H], norm_b[2 * H:2 * H + D], norm_b[2 * H + D:],
        fc1_W[:H], fc1_W[H:2 * H], fc1_W[2 * H:2 * H + D], fc1_W[2 * H + D:],
        fc1_b, fc2_W, fc2_b, out_W, out_b)


# speedup vs baseline: 2.5468x; 1.0039x over previous
import functools

import jax
import jax.numpy as jnp
from jax import lax
from jax.experimental import pallas as pl
from jax.experimental.pallas import tpu as pltpu
from jax.experimental.pallas import tpu_sc as plsc

N = 10000
E = 320000
F_IN = 128
H = 256
B = 64
D = 16
EPS = 1e-5

BLK = 1000
NB = N // BLK
NC = 2
NS = 16
CHUNK = 80
NBUF = 4
ZROWS = 624
ZTAIL = N - NS * ZROWS


def _sc_scatter_body(split_features, table, srcx, dst, zeros, out, *refs):
    sv = refs[0:NBUF]
    dv = refs[NBUF:2 * NBUF]
    rows = refs[2 * NBUF:3 * NBUF]
    acc = refs[3 * NBUF]
    isem = refs[3 * NBUF + 1:3 * NBUF + 3]
    gsem = refs[3 * NBUF + 3:3 * NBUF + 5]
    ssem = refs[3 * NBUF + 5:3 * NBUF + 7]
    c = lax.axis_index("c")
    s = lax.axis_index("s")
    r0 = s * ZROWS
    pltpu.sync_copy(zeros.at[pl.ds(0, ZROWS)], acc.at[pl.ds(r0, ZROWS)])

    @pl.when(s == 0)
    def _():
        pltpu.sync_copy(zeros.at[pl.ds(0, ZTAIL)],
                        acc.at[pl.ds(NS * ZROWS, ZTAIL)])

    plsc.subcore_barrier()

    if split_features:
        ne = E // NS
        src_base = c * E + s * ne
        dst_base = s * ne
    else:
        ne = E // (NC * NS)
        w = s * NC + c
        src_base = w * ne
        dst_base = w * ne
    nch = ne // CHUNK
    ngrp = nch // NBUF
    tail = nch % NBUF
    HALF = NBUF // 2

    def half_step(h, j0, k, wait_prev):
        bs = list(range(h * HALF, h * HALF + k))
        if wait_prev:
            for b in range(h * HALF, (h + 1) * HALF):
                pltpu.make_async_copy(rows[b], acc.at[dv[b]], ssem[h]).wait()
        for i, b in enumerate(bs):
            pltpu.async_copy(
                srcx.at[pl.ds(src_base + (j0 + i) * CHUNK, CHUNK)], sv[b], isem[h])
            pltpu.async_copy(
                dst.at[pl.ds(dst_base + (j0 + i) * CHUNK, CHUNK)], dv[b], isem[h])
        for b in bs:
            pltpu.make_async_copy(srcx.at[pl.ds(src_base, CHUNK)], sv[b], isem[h]).wait()
            pltpu.make_async_copy(dst.at[pl.ds(dst_base, CHUNK)], dv[b], isem[h]).wait()
        for b in bs:
            pltpu.async_copy(table.at[sv[b]], rows[b], gsem[h])
        for b in bs:
            pltpu.make_async_copy(table.at[sv[b]], rows[b], gsem[h]).wait()
        for b in bs:
            pltpu.async_copy(rows[b], acc.at[dv[b]], ssem[h], add=True)

    half_step(0, 0, HALF, False)
    half_step(1, HALF, HALF, False)

    def body(g, carry):
        j0 = g * NBUF
        half_step(0, j0, HALF, True)
        half_step(1, j0 + HALF, HALF, True)
        return carry

    lax.fori_loop(1, ngrp, body, 0)
    if tail:
        half_step(0, ngrp * NBUF, tail, True)
    for b in range(tail if tail else HALF):
        pltpu.make_async_copy(rows[b], acc.at[dv[b]], ssem[0]).wait()
    for b in range(HALF, NBUF):
        pltpu.make_async_copy(rows[b], acc.at[dv[b]], ssem[1]).wait()
    plsc.subcore_barrier()
    pltpu.sync_copy(acc.at[pl.ds(r0, ZROWS)],
                    out.at[pl.ds(c * N + r0, ZROWS)])

    @pl.when(s == 0)
    def _():
        pltpu.sync_copy(acc.at[pl.ds(NS * ZROWS, ZTAIL)],
                        out.at[pl.ds(c * N + NS * ZROWS, ZTAIL)])


def _make_sc_scatter(split_features):
    mesh = plsc.VectorSubcoreMesh(core_axis_name="c", subcore_axis_name="s",
                                  num_cores=NC, num_subcores=NS)
    return pl.kernel(
        functools.partial(_sc_scatter_body, split_features),
        out_type=jax.ShapeDtypeStruct((NC * N, 128), jnp.float32),
        mesh=mesh,
        scratch_types=(
            [pltpu.VMEM((CHUNK,), jnp.int32) for _ in range(2 * NBUF)]
            + [pltpu.VMEM((CHUNK, 128), jnp.float32) for _ in range(NBUF)]
            + [pltpu.VMEM_SHARED((N, 128), jnp.float32)]
            + [pltpu.SemaphoreType.DMA for _ in range(6)]
        ),
    )


def _seg_stats(i, u, b_vec, st_ref):
    oh = (b_vec[None, :] == lax.broadcasted_iota(jnp.int32, (B, BLK), 0)
          ).astype(jnp.float32)
    r1 = jnp.sum(u, axis=1)
    r2 = jnp.sum(u * u, axis=1)

    @pl.when(i == 0)
    def _():
        st_ref[...] = jnp.zeros_like(st_ref)

    st_ref[0, :] += jnp.sum(oh * r1[None, :], axis=1)
    st_ref[1, :] += jnp.sum(oh * r2[None, :], axis=1)
    st_ref[2, :] += jnp.sum(oh, axis=1)


def _t1a_body(x_ref, p0_ref, p1_ref, b_ref, Wa_ref, ba_ref, Wb_ref, bb_ref,
              u_ref, st_ref):
    i = pl.program_id(0)
    y = x_ref[...] + p0_ref[...] + p1_ref[...]
    t = jnp.dot(y, Wa_ref[...], preferred_element_type=jnp.float32)
    t = jnp.maximum(t + ba_ref[...][None, :], 0.0)
    u = jnp.dot(t, Wb_ref[...], preferred_element_type=jnp.float32)
    u = u + bb_ref[...][None, :]
    u_ref[...] = u
    _seg_stats(i, u, b_ref[0, 0], st_ref)


def _t2a_body(h0_ref, h1_ref, a0_ref, a1_ref, b_ref, Wa_ref,
              ba_ref, Wb_ref, bb_ref, u_ref, st_ref):
    i = pl.program_id(0)
    y0 = h0_ref[...] + a0_ref[...]
    y1 = h1_ref[...] + a1_ref[...]
    y = jnp.concatenate([y0, y1], axis=1)
    t = jnp.dot(y, Wa_ref[...], preferred_element_type=jnp.float32)
    t = jnp.maximum(t + ba_ref[...][None, :], 0.0)
    u = jnp.dot(t, Wb_ref[...], preferred_element_type=jnp.float32)
    u = u + bb_ref[...][None, :]
    u_ref[...] = u
    _seg_stats(i, u, b_ref[0, 0], st_ref)


def _graph_ln(u, b_vec, st_ref, w_ref, bias_ref):
    deg = st_ref[2, :]
    norm = jnp.maximum(deg, 1.0) * H
    mean_g = st_ref[0, :] / norm
    var_g = st_ref[1, :] / norm - mean_g * mean_g
    inv_g = 1.0 / jnp.sqrt(var_g + EPS)
    ohT = (b_vec[:, None] == lax.broadcasted_iota(jnp.int32, (BLK, B), 1)
           ).astype(jnp.float32)
    mean_n = jnp.dot(ohT, mean_g[:, None], preferred_element_type=jnp.float32,
                     precision=lax.Precision.HIGHEST)
    inv_n = jnp.dot(ohT, inv_g[:, None], preferred_element_type=jnp.float32,
                    precision=lax.Precision.HIGHEST)
    out = (u - mean_n) * inv_n * w_ref[...][None, :] + bias_ref[...][None, :]
    return jnp.maximum(out, 0.0)


def _t1b_body(u_ref, st_ref, b_ref, w_ref, bias_ref, h_ref):
    out = _graph_ln(u_ref[...], b_ref[0, 0], st_ref, w_ref, bias_ref)
    h_ref[0] = out[:, :128]
    h_ref[1] = out[:, 128:]


def _t2b_body(u_ref, st_ref, b_ref, w_ref, bias_ref, add_ref):
    i = pl.program_id(0)
    out = _graph_ln(u_ref[...], b_ref[0, 0], st_ref, w_ref, bias_ref)
    b_vec = b_ref[0, 0]
    oh = (b_vec[None, :] == lax.broadcasted_iota(jnp.int32, (B, BLK), 0)
          ).astype(jnp.float32)

    @pl.when(i == 0)
    def _():
        add_ref[...] = jnp.zeros_like(add_ref)

    add_ref[...] += jnp.dot(oh, out, preferred_element_type=jnp.float32,
                            precision=lax.Precision.HIGHEST)


def _t3_body(a1_ref, st1_ref, a2_ref, st2_ref, d1_ref, d2_ref,
             nw1, nw2, nw3, nw4, nb1, nb2, nb3, nb4,
             f1A, f1B, f1C, f1D, f1b_ref, f2W_ref, f2b_ref,
             oW_ref, ob_ref, out_ref):
    IN = 2 * H + 2 * D
    deg1 = jnp.maximum(st1_ref[2, :], 1.0)[:, None]
    emb1 = a1_ref[...] * (1.0 + 1.0 / deg1)
    deg2 = jnp.maximum(st2_ref[2, :], 1.0)[:, None]
    emb2 = a2_ref[...] * (1.0 + 1.0 / deg2)
    p1 = emb1 + emb2
    p2 = jnp.abs(emb1 - emb2)
    p3 = d1_ref[...] + d2_ref[...]
    p4 = jnp.abs(d1_ref[...] - d2_ref[...])
    s = (jnp.sum(p1, axis=1) + jnp.sum(p2, axis=1)
         + jnp.sum(p3, axis=1) + jnp.sum(p4, axis=1))
    mu = s / IN
    q = (jnp.sum(p1 * p1, axis=1) + jnp.sum(p2 * p2, axis=1)
         + jnp.sum(p3 * p3, axis=1) + jnp.sum(p4 * p4, axis=1))
    var = q / IN - mu * mu
    inv = (1.0 / jnp.sqrt(var + EPS))[:, None]
    mu = mu[:, None]
    c1 = (p1 - mu) * inv * nw1[...][None, :] + nb1[...][None, :]
    c2 = (p2 - mu) * inv * nw2[...][None, :] + nb2[...][None, :]
    c3 = (p3 - mu) * inv * nw3[...][None, :] + nb3[...][None, :]
    c4 = (p4 - mu) * inv * nw4[...][None, :] + nb4[...][None, :]
    z = (jnp.dot(c1, f1A[...], preferred_element_type=jnp.float32)
         + jnp.dot(c2, f1B[...], preferred_element_type=jnp.float32)
         + jnp.dot(c3, f1C[...], preferred_element_type=jnp.float32)
         + jnp.dot(c4, f1D[...], preferred_element_type=jnp.float32))
    z = jnp.maximum(z + f1b_ref[...][None, :], 0.0)
    z = jnp.dot(z, f2W_ref[...], preferred_element_type=jnp.float32)
    z = jnp.maximum(z + f2b_ref[...][None, :], 0.0)
    z = jnp.dot(z, oW_ref[...], preferred_element_type=jnp.float32)
    out_ref[...] = z + ob_ref[...][None, :]


def _build():
    fns = {}

    def row_blk(off):
        return pl.BlockSpec((BLK, 128), lambda i, off=off: (off + i, 0))

    batch_spec = pl.BlockSpec((1, 1, BLK), lambda i: (i, 0, 0))

    def full(shape):
        return pl.BlockSpec(shape, lambda i, n=len(shape): (0,) * n)

    st_spec = pl.BlockSpec((8, B), lambda i: (0, 0))

    fns["t1a"] = pl.pallas_call(
        _t1a_body,
        grid=(NB,),
        in_specs=[
            pl.BlockSpec((BLK, F_IN), lambda i: (i, 0)),
            row_blk(0), row_blk(NB),
            batch_spec,
            full((F_IN, H)), full((H,)), full((H, H)), full((H,)),
        ],
        out_specs=(pl.BlockSpec((BLK, H), lambda i: (i, 0)), st_spec),
        out_shape=(jax.ShapeDtypeStruct((N, H), jnp.float32),
                   jax.ShapeDtypeStruct((8, B), jnp.float32)),
    )
    fns["t2a"] = pl.pallas_call(
        _t2a_body,
        grid=(NB,),
        in_specs=[
            row_blk(0), row_blk(NB), row_blk(0), row_blk(NB),
            batch_spec,
            full((H, H)),
            full((H,)), full((H, H)), full((H,)),
        ],
        out_specs=(pl.BlockSpec((BLK, H), lambda i: (i, 0)), st_spec),
        out_shape=(jax.ShapeDtypeStruct((N, H), jnp.float32),
                   jax.ShapeDtypeStruct((8, B), jnp.float32)),
    )
    fns["t1b"] = pl.pallas_call(
        _t1b_body,
        grid=(NB,),
        in_specs=[
            pl.BlockSpec((BLK, H), lambda i: (i, 0)),
            st_spec, batch_spec, full((H,)), full((H,)),
        ],
        out_specs=pl.BlockSpec((2, BLK, 128), lambda i: (0, i, 0)),
        out_shape=jax.ShapeDtypeStruct((2, N, 128), jnp.float32),
    )
    fns["t2b"] = pl.pallas_call(
        _t2b_body,
        grid=(NB,),
        in_specs=[
            pl.BlockSpec((BLK, H), lambda i: (i, 0)),
            st_spec, batch_spec, full((H,)), full((H,)),
        ],
        out_specs=pl.BlockSpec((B, H), lambda i: (0, 0)),
        out_shape=jax.ShapeDtypeStruct((B, H), jnp.float32),
    )
    fns["t3"] = pl.pallas_call(
        _t3_body,
        out_shape=jax.ShapeDtypeStruct((B, 1), jnp.float32),
    )
    return fns


_FNS = _build()


def _get_sc(name):
    fn = _FNS.get(name)
    if fn is None:
        fn = _FNS[name] = _make_sc_scatter(name == "sc2")
    return fn


def _backbone(x, src, dst, batch3, zeros,
              W1a, b1a, W1b, b1b, ln1_w, ln1_b,
              W2a, b2a, W2b, b2b, ln2_w, ln2_b):
    p = _get_sc("sc1")(x, src, dst, zeros)
    u1, st1 = _FNS["t1a"](x, p, p, batch3, W1a, b1a, W1b, b1b)
    h = _FNS["t1b"](u1, st1, batch3, ln1_w, ln1_b)
    h2 = h.reshape(NC * N, 128)
    src2 = jnp.concatenate([src, src + N], axis=0)
    a = _get_sc("sc2")(h2, src2, dst, zeros)
    u2, st2 = _FNS["t2a"](h2, h2, a, a, batch3, W2a, b2a, W2b, b2b)
    add = _FNS["t2b"](u2, st2, batch3, ln2_w, ln2_b)
    return add, st2


def kernel(g1_x, g1_edge_index, g1_batch, g2_x, g2_edge_index, g2_batch,
           d1, d2, W1a, b1a, W1b, b1b, ln1_w, ln1_b, W2a, b2a, W2b, b2b,
           ln2_w, ln2_b, norm_w, norm_b, fc1_W, fc1_b, fc2_W, fc2_b,
           out_W, out_b):
    zeros = jnp.zeros((ZROWS, 128), jnp.float32)
    bb = (W1a, b1a, W1b, b1b, ln1_w, ln1_b, W2a, b2a, W2b, b2b, ln2_w, ln2_b)
    add1, st1 = _backbone(g1_x, g1_edge_index[0], g1_edge_index[1],
                          g1_batch.reshape(NB, 1, BLK), zeros, *bb)
    add2, st2 = _backbone(g2_x, g2_edge_index[0], g2_edge_index[1],
                          g2_batch.reshape(NB, 1, BLK), zeros, *bb)
    return _FNS["t3"](
        add1, st1, add2, st2, d1, d2,
        norm_w[:H], norm_w[H:2 * H], norm_w[2 * H:2 * H + D], norm_w[2 * H + D:],
        norm_b[:H], norm_b[H:2 * H], norm_b[2 * H:2 * H + D], norm_b[2 * H + D:],
        fc1_W[:H], fc1_W[H:2 * H], fc1_W[2 * H:2 * H + D], fc1_W[2 * H + D:],
        fc1_b, fc2_W, fc2_b, out_W, out_b)
